# Initial kernel scaffold; baseline (speedup 1.0000x reference)
import jax
import jax.numpy as jnp
from jax.experimental import pallas as pl


def kernel(predictions, actions_label):
    raise NotImplementedError("write your pallas kernel here")


# speedup vs baseline: 1.3606x; 1.3606x over previous
import functools

import jax
import jax.numpy as jnp
from jax.experimental import pallas as pl

B = 128
C = 64
T = 4096


def _argmax_onehot(x):
    m = jnp.max(x, axis=0, keepdims=True)
    ci = jax.lax.broadcasted_iota(jnp.int32, x.shape, 0)
    cand = jnp.where(x == m, ci, C)
    idx = jnp.min(cand, axis=0, keepdims=True)
    oh = (ci == idx).astype(jnp.float32)
    return idx, oh


def _adj_normalized(x):
    idx, oh = _argmax_onehot(x)
    nxt_idx = jnp.roll(idx, -1, axis=1)
    nxt_oh = jnp.roll(oh, -1, axis=1)
    t_iota = jax.lax.broadcasted_iota(jnp.int32, idx.shape, 1)
    valid = ((idx != nxt_idx) & (t_iota < T - 1)).astype(jnp.float32)
    cur = (oh * valid).astype(jnp.bfloat16)
    counts = jax.lax.dot_general(
        cur, nxt_oh.astype(jnp.bfloat16),
        (((1,), (1,)), ((), ())),
        preferred_element_type=jnp.float32)
    row = jnp.sum(counts, axis=1, keepdims=True)
    return counts, counts / (row + 1e-8)


def _body(pred_ref, act_ref, bce_ref, sq_ref, cnt_ref):
    b = pl.program_id(0)

    t_counts, true_adj = _adj_normalized(act_ref[0])
    _, pred_adj = _adj_normalized(pred_ref[0])

    gt = (t_counts > 0).astype(jnp.float32)
    dense_pred = jnp.tanh(pred_adj)
    log_p = jnp.maximum(jnp.log(dense_pred), -100.0)
    log_1mp = jnp.maximum(jnp.log1p(-dense_pred), -100.0)
    bce = -jnp.sum(gt * log_p + (1.0 - gt) * log_1mp,
                   axis=(0, 1), keepdims=True)

    sq = (pred_adj - true_adj) ** 2
    sqs = jnp.sum(gt * sq, axis=(0, 1), keepdims=True)
    cnts = jnp.sum(gt, axis=(0, 1), keepdims=True)

    @pl.when(b == 0)
    def _():
        bce_ref[:, :] = bce
        sq_ref[:, :] = sqs
        cnt_ref[:, :] = cnts

    @pl.when(b != 0)
    def _():
        bce_ref[:, :] += bce
        sq_ref[:, :] += sqs
        cnt_ref[:, :] += cnts


@functools.partial(jax.jit)
def kernel(predictions, actions_label):
    scalar = jax.ShapeDtypeStruct((1, 1), jnp.float32)
    bce_sum, sq_sum, cnt_sum = pl.pallas_call(
        _body,
        grid=(B,),
        in_specs=[
            pl.BlockSpec((1, C, T), lambda b: (b, 0, 0)),
            pl.BlockSpec((1, C, T), lambda b: (b, 0, 0)),
        ],
        out_specs=[
            pl.BlockSpec((1, 1), lambda b: (0, 0)),
            pl.BlockSpec((1, 1), lambda b: (0, 0)),
            pl.BlockSpec((1, 1), lambda b: (0, 0)),
        ],
        out_shape=[scalar, scalar, scalar],
    )(predictions, actions_label)

    bce = bce_sum[0, 0] / (B * C * C)
    cnt = cnt_sum[0, 0]
    mse = sq_sum[0, 0] / jnp.maximum(cnt, 1.0)
    return bce + jnp.where(cnt > 0, mse, 0.0)


# speedup vs baseline: 1.6412x; 1.2062x over previous
import functools

import jax
import jax.numpy as jnp
from jax.experimental import pallas as pl

B = 128
C = 64
T = 4096


def _trans_counts(x, lane_mask, offdiag):
    m = jnp.max(x, axis=0, keepdims=True)
    oh = (x == m).astype(jnp.float32).astype(jnp.bfloat16)
    nxt = jnp.roll(oh, -1, axis=1)
    cur = oh * lane_mask
    counts = jax.lax.dot_general(
        cur, nxt, (((1,), (1,)), ((), ())),
        preferred_element_type=jnp.float32)
    return counts * offdiag


def _body(pred_ref, act_ref, bce_ref, sq_ref, cnt_ref):
    b = pl.program_id(0)

    t_iota = jax.lax.broadcasted_iota(jnp.int32, (1, T), 1)
    lane_mask = (t_iota < T - 1).astype(jnp.bfloat16)
    ri = jax.lax.broadcasted_iota(jnp.int32, (C, C), 0)
    cj = jax.lax.broadcasted_iota(jnp.int32, (C, C), 1)
    offdiag = (ri != cj).astype(jnp.float32)

    t_counts = _trans_counts(act_ref[0], lane_mask, offdiag)
    p_counts = _trans_counts(pred_ref[0], lane_mask, offdiag)

    true_adj = t_counts / (jnp.sum(t_counts, axis=1, keepdims=True) + 1e-8)
    pred_adj = p_counts / (jnp.sum(p_counts, axis=1, keepdims=True) + 1e-8)

    gt = (t_counts > 0).astype(jnp.float32)
    dense_pred = jnp.tanh(pred_adj)
    log_p = jnp.maximum(jnp.log(dense_pred), -100.0)
    log_1mp = jnp.maximum(jnp.log1p(-dense_pred), -100.0)
    bce = -jnp.sum(gt * log_p + (1.0 - gt) * log_1mp,
                   axis=(0, 1), keepdims=True)

    sq = (pred_adj - true_adj) ** 2
    sqs = jnp.sum(gt * sq, axis=(0, 1), keepdims=True)
    cnts = jnp.sum(gt, axis=(0, 1), keepdims=True)

    @pl.when(b == 0)
    def _():
        bce_ref[:, :] = bce
        sq_ref[:, :] = sqs
        cnt_ref[:, :] = cnts

    @pl.when(b != 0)
    def _():
        bce_ref[:, :] += bce
        sq_ref[:, :] += sqs
        cnt_ref[:, :] += cnts


@functools.partial(jax.jit)
def kernel(predictions, actions_label):
    scalar = jax.ShapeDtypeStruct((1, 1), jnp.float32)
    bce_sum, sq_sum, cnt_sum = pl.pallas_call(
        _body,
        grid=(B,),
        in_specs=[
            pl.BlockSpec((1, C, T), lambda b: (b, 0, 0)),
            pl.BlockSpec((1, C, T), lambda b: (b, 0, 0)),
        ],
        out_specs=[
            pl.BlockSpec((1, 1), lambda b: (0, 0)),
            pl.BlockSpec((1, 1), lambda b: (0, 0)),
            pl.BlockSpec((1, 1), lambda b: (0, 0)),
        ],
        out_shape=[scalar, scalar, scalar],
    )(predictions, actions_label)

    bce = bce_sum[0, 0] / (B * C * C)
    cnt = cnt_sum[0, 0]
    mse = sq_sum[0, 0] / jnp.maximum(cnt, 1.0)
    return bce + jnp.where(cnt > 0, mse, 0.0)
